# baseline (speedup 1.0000x reference)
import jax
import jax.numpy as jnp
from jax.experimental import pallas as pl
from jax.experimental.pallas import tpu as pltpu

REF_TOKEN_ID = 32000
N0_OFF = 257
R_MAX = 16


def _index_kernel(ids_ref, idx_ref, valid_ref):
    ids = ids_ref[...]
    B, S = ids.shape
    pos = jax.lax.broadcasted_iota(jnp.int32, (B, S), 1)
    mask = (ids == REF_TOKEN_ID) & (pos >= 1)
    mi = mask.astype(jnp.int32)
    cum = mi
    k = 1
    while k < S:
        shifted = jnp.concatenate(
            [jnp.zeros((B, k), jnp.int32), cum[:, : S - k]], axis=1)
        cum = cum + shifted
        k *= 2
    count = cum[:, S - 1:S]
    idx_cols = []
    for r in range(R_MAX):
        sel = mask & (cum == (r + 1))
        idx_r = jnp.sum(jnp.where(sel, pos, 0), axis=1, keepdims=True)
        idx_cols.append(idx_r)
    idx = jnp.concatenate(idx_cols, axis=1) + (N0_OFF - 1)
    boff = jax.lax.broadcasted_iota(jnp.int32, (B, R_MAX), 0) * (N0_OFF + S)
    rr = jax.lax.broadcasted_iota(jnp.int32, (B, R_MAX), 1)
    valid_ref[...] = (rr < count).astype(jnp.int32)
    idx_ref[...] = idx + boff


def _proj_kernel(idx_ref, valid_ref, x_ref, w_ref, b_ref, out_ref):
    b = pl.program_id(0)
    r = pl.program_id(1)
    x = x_ref[0]
    y = jnp.dot(x, w_ref[...], preferred_element_type=jnp.float32)
    y = y + b_ref[...]
    v = valid_ref[b, r]
    out_ref[0] = jnp.where(v > 0, y, 0.0)


def kernel(input_ids, last_hidden_state, W_proj, b_proj):
    B, S = input_ids.shape
    _, L, D = last_hidden_state.shape
    DG = W_proj.shape[1]
    ids32 = input_ids.astype(jnp.int32)

    idx, valid = pl.pallas_call(
        _index_kernel,
        out_shape=(
            jax.ShapeDtypeStruct((B, R_MAX), jnp.int32),
            jax.ShapeDtypeStruct((B, R_MAX), jnp.int32),
        ),
    )(ids32)

    hs_flat = last_hidden_state.reshape(B * L, 1, D)
    grid_spec = pltpu.PrefetchScalarGridSpec(
        num_scalar_prefetch=2,
        grid=(B, R_MAX),
        in_specs=[
            pl.BlockSpec((1, 1, D), lambda b, r, idx_ref, valid_ref: (idx_ref[b, r], 0, 0)),
            pl.BlockSpec((D, DG), lambda b, r, idx_ref, valid_ref: (0, 0)),
            pl.BlockSpec((1, DG), lambda b, r, idx_ref, valid_ref: (0, 0)),
        ],
        out_specs=pl.BlockSpec((1, 1, DG), lambda b, r, idx_ref, valid_ref: (b * R_MAX + r, 0, 0)),
    )
    out = pl.pallas_call(
        _proj_kernel,
        grid_spec=grid_spec,
        out_shape=jax.ShapeDtypeStruct((B * R_MAX, 1, DG), jnp.float32),
    )(idx, valid, hs_flat, W_proj, b_proj.reshape(1, DG))
    return out.reshape(B, R_MAX, DG)


# speedup vs baseline: 3.4318x; 3.4318x over previous
import jax
import jax.numpy as jnp
from jax.experimental import pallas as pl
from jax.experimental.pallas import tpu as pltpu
from jax.experimental.pallas import tpu_sc as plsc

REF_TOKEN_ID = 32000
SEG_OFF = 256
R_MAX = 16
LANES = 16


def _sc_gather_body(ids_hbm, hs_hbm, gath_hbm, valid_hbm,
                    ids_v, idx_v, tmp_v, rows_v, sem):
    B, S = ids_hbm.shape
    cid = jax.lax.axis_index("c")
    sid = jax.lax.axis_index("s")
    nc = 2
    wid = sid * nc + cid

    @pl.when(wid < B)
    def _():
        b = wid
        pltpu.sync_copy(ids_hbm.at[b], ids_v)
        idx_v[...] = jnp.zeros((LANES,), jnp.int32)
        lane = jax.lax.iota(jnp.int32, LANES)

        ref_tok = jnp.full((LANES,), REF_TOKEN_ID, jnp.int32)
        ones_v = jnp.full((LANES,), 1, jnp.int32)
        rmax_v = jnp.full((LANES,), R_MAX, jnp.int32)
        seg_v = jnp.full((LANES,), SEG_OFF, jnp.int32)

        def chunk(j, cnt):
            v = ids_v[pl.ds(j * LANES, LANES)]
            pos = jnp.full((LANES,), j * LANES, jnp.int32) + lane
            m = (v == ref_tok) & (pos >= ones_v)
            mi = jnp.where(m, ones_v, 0)
            csum = plsc.cumsum(mi)
            tgt = jnp.full((LANES,), cnt, jnp.int32) + csum - ones_v
            m2 = m & (tgt < rmax_v)
            plsc.store_scatter(idx_v, [tgt], pos + seg_v, mask=m2)
            return cnt + jnp.sum(mi)

        cnt = jax.lax.fori_loop(0, S // LANES, chunk, jnp.int32(0))
        pltpu.async_copy(hs_hbm.at[b].at[idx_v], rows_v, sem).wait()
        pltpu.sync_copy(rows_v, gath_hbm.at[pl.ds(b * R_MAX, R_MAX)])
        cnt_v = jnp.full((LANES,), cnt, jnp.int32)
        tmp_v[...] = jnp.where(lane < cnt_v, ones_v, 0)
        pltpu.sync_copy(tmp_v, valid_hbm.at[pl.ds(b * R_MAX, R_MAX)])


def _proj_body(g_ref, w_ref, b_ref, v_ref, out_ref):
    x = g_ref[...]
    y = jnp.dot(x, w_ref[...], preferred_element_type=jnp.float32)
    y = y + b_ref[...]
    m = v_ref[...] > 0
    out_ref[0] = jnp.where(m, y, 0.0)


def kernel(input_ids, last_hidden_state, W_proj, b_proj):
    B, S = input_ids.shape
    _, L, D = last_hidden_state.shape
    DG = W_proj.shape[1]
    ids32 = input_ids.astype(jnp.int32)

    mesh = plsc.VectorSubcoreMesh(core_axis_name="c", subcore_axis_name="s")
    sc_call = pl.kernel(
        _sc_gather_body,
        out_type=(
            jax.ShapeDtypeStruct((B * R_MAX, D), jnp.float32),
            jax.ShapeDtypeStruct((B * R_MAX,), jnp.int32),
        ),
        mesh=mesh,
        compiler_params=pltpu.CompilerParams(needs_layout_passes=False),
        scratch_types=[
            pltpu.VMEM((S,), jnp.int32),
            pltpu.VMEM((LANES,), jnp.int32),
            pltpu.VMEM((LANES,), jnp.int32),
            pltpu.VMEM((R_MAX, D), jnp.float32),
            pltpu.SemaphoreType.DMA,
        ],
    )
    gathered, valid = sc_call(ids32, last_hidden_state)

    out = pl.pallas_call(
        _proj_body,
        grid=(B,),
        in_specs=[
            pl.BlockSpec((R_MAX, D), lambda b: (b, 0)),
            pl.BlockSpec((D, DG), lambda b: (0, 0)),
            pl.BlockSpec((1, DG), lambda b: (0, 0)),
            pl.BlockSpec((R_MAX, 1), lambda b: (b, 0)),
        ],
        out_specs=pl.BlockSpec((1, R_MAX, DG), lambda b: (b, 0, 0)),
        out_shape=jax.ShapeDtypeStruct((B, R_MAX, DG), jnp.float32),
    )(gathered, W_proj, b_proj.reshape(1, DG), valid.reshape(B * R_MAX, 1))
    return out
